# baseline (speedup 1.0000x reference)
import functools

import jax
import jax.numpy as jnp
import numpy as np
from jax import lax
from jax.experimental import pallas as pl
from jax.experimental.pallas import tpu as pltpu
from jax.experimental.pallas import tpu_sc as plsc

N_PTS = 524288
NLVL = 16
LDIM = 2
TBL = 2 ** 19
NBINS = 16
_SCALE = float(np.exp2(np.log2(512 / 16) / (NLVL - 1)))
RES_LIST = [int(np.ceil(16 * _SCALE ** l)) for l in range(NLVL)]
TILED_LIST = [(r + 1) ** 3 <= TBL for r in RES_LIST]
HK1 = 2654435761
HK2 = 805459861
NW = 32
PW = N_PTS // NW
BLK = 512
LOC_LVLS = 4
_loc_sz = [((RES_LIST[l] + 1) ** 3 + 15) & ~15 for l in range(LOC_LVLS)]
LOC_OFF = [sum(_loc_sz[:l]) for l in range(LOC_LVLS)]
LOC_TOTAL = sum(_loc_sz)


def _index_body(pts_ref, out_ref):
    p = pts_ref[...]
    x = jnp.clip((p + 1.0) * 0.5, 0.0, 1.0 - 1e-6)
    for l in range(NLVL):
        res = RES_LIST[l]
        pos = x * jnp.float32(res)
        p0 = jnp.floor(pos).astype(jnp.uint32)
        cx = p0[0:1, :]
        cy = p0[1:2, :]
        cz = p0[2:3, :]
        base = jnp.uint32(LOC_OFF[l] if l < LOC_LVLS else l * TBL)
        for c in range(8):
            if TILED_LIST[l]:
                s = res + 1
                ox, oy, oz = c & 1, (c >> 1) & 1, (c >> 2) & 1
                idx = ((cx + jnp.uint32(ox))
                       + (cy + jnp.uint32(oy)) * jnp.uint32(s)
                       + (cz + jnp.uint32(oz)) * jnp.uint32(s * s) + base)
            else:
                hx = cx + jnp.uint32(c & 1)
                hy = cy * jnp.uint32(HK1) + jnp.uint32(HK1 if (c >> 1) & 1 else 0)
                hz = cz * jnp.uint32(HK2) + jnp.uint32(HK2 if (c >> 2) & 1 else 0)
                idx = ((hx ^ hy ^ hz) & jnp.uint32(TBL - 1)) + base
            out_ref[0:1, l:l + 1, c * BLK:(c + 1) * BLK] = (
                idx.astype(jnp.int32).reshape(1, 1, BLK))


def _tc_indices(pts_t):
    return pl.pallas_call(
        _index_body,
        grid=(N_PTS // BLK,),
        in_specs=[pl.BlockSpec((3, BLK), lambda i: (0, i))],
        out_specs=pl.BlockSpec((1, NLVL, 8 * BLK), lambda i: (i, 0, 0)),
        out_shape=jax.ShapeDtypeStruct((N_PTS // BLK, NLVL, 8 * BLK), jnp.int32),
    )(pts_t)


def _sc_embed(grid_packed, idx_all, pts_t):
    mesh = plsc.VectorSubcoreMesh(core_axis_name="c", subcore_axis_name="s")
    NGRP = BLK // 16
    NB = PW // BLK

    @functools.partial(
        pl.kernel,
        mesh=mesh,
        compiler_params=pltpu.CompilerParams(
            needs_layout_passes=False, use_tc_tiling_on_sc=False),
        out_type=jax.ShapeDtypeStruct((2 * NLVL, N_PTS), jnp.float32),
        scratch_types=[
            [pltpu.VMEM((2 * BLK,), jnp.int32) for _ in range(8)],
            [pltpu.VMEM((2 * BLK,), jnp.int32) for _ in range(8)],
            pltpu.VMEM((3, BLK), jnp.float32),
            pltpu.VMEM((2 * NLVL, BLK), jnp.float32),
            pltpu.VMEM((LOC_TOTAL,), jnp.int32),
            [pltpu.SemaphoreType.DMA for _ in range(2)],
            [pltpu.SemaphoreType.DMA for _ in range(2)],
        ],
    )
    def k(grid_hbm, idx_hbm, pts_hbm, out_hbm, idxb, crnb, xv, acc_v,
          tab_v, isem, gsem):
        wid = lax.axis_index("s") * 2 + lax.axis_index("c")
        base_pt = wid * PW
        wb0 = wid * NB
        hi_mask = jnp.int32(-65536)

        class _Multi:
            def __init__(self, hs):
                self.hs = hs

            def start(self):
                for h in self.hs:
                    h.start()

            def wait(self):
                for h in self.hs:
                    h.wait()

        def idx_cp(l, gb, j):
            return _Multi([
                pltpu.make_async_copy(
                    idx_hbm.at[gb, l, pl.ds(q * 2 * BLK, 2 * BLK)],
                    idxb[j * 4 + q], isem[j])
                for q in range(4)
            ])

        def gath(j):
            return _Multi([
                pltpu.make_async_copy(grid_hbm.at[idxb[j * 4 + q]],
                                      crnb[j * 4 + q], gsem[j])
                for q in range(4)
            ])

        idx_cp(0, wb0, 0).start()
        idx_cp(1, wb0, 1).start()
        for l in range(LOC_LVLS):
            pltpu.sync_copy(grid_hbm.at[pl.ds(l * TBL, _loc_sz[l])],
                            tab_v.at[pl.ds(LOC_OFF[l], _loc_sz[l])])
        idx_cp(0, wb0, 0).wait()

        def block_body(b, carry):
            p0 = base_pt + b * BLK
            gb = wb0 + b
            not_last = b < NB - 1
            pltpu.sync_copy(pts_hbm.at[:, pl.ds(p0, BLK)], xv)
            for l in range(NLVL):
                j = l & 1
                jn = 1 - j
                local = l < LOC_LVLS
                res_f = jnp.float32(RES_LIST[l])
                if not local:
                    gath(j).wait()
                l1 = (l + 1) % NLVL
                gb1 = gb + (l + 1) // NLVL
                if l < NLVL - 1:
                    idx_cp(l1, gb1, jn).wait()
                    if l1 >= LOC_LVLS:
                        gath(jn).start()
                else:
                    @pl.when(not_last)
                    def _():
                        idx_cp(l1, gb1, jn).wait()
                l2 = (l + 2) % NLVL
                gb2 = gb + (l + 2) // NLVL
                if not local:
                    if l < NLVL - 2:
                        idx_cp(l2, gb2, j).start()
                    else:
                        @pl.when(not_last)
                        def _():
                            idx_cp(l2, gb2, j).start()

                crn4 = [crnb[j * 4 + q] for q in range(4)]
                idx4 = [idxb[j * 4 + q] for q in range(4)]

                def grp(g, c2):
                    for u in range(2):
                        sl = pl.ds((2 * g + u) * 16, 16)
                        fr = []
                        for d in range(3):
                            xe = xv[d, sl]
                            pos = jnp.minimum(
                                jnp.maximum(xe * 0.5 + 0.5, 0.0),
                                1.0 - 1e-6) * res_f
                            fr.append(pos - lax.convert_element_type(
                                lax.convert_element_type(pos, jnp.int32),
                                jnp.float32))
                        fx, fy, fz = fr
                        va = []
                        vb = []
                        for c in range(8):
                            if local:
                                iv = idx4[c // 2][
                                    pl.ds((c & 1) * BLK + (2 * g + u) * 16, 16)]
                                w = plsc.load_gather(tab_v, [iv])
                            else:
                                w = crn4[c // 2][
                                    pl.ds((c & 1) * BLK + (2 * g + u) * 16, 16)]
                            va.append(plsc.bitcast(lax.shift_left(w, 16),
                                                   jnp.float32))
                            vb.append(plsc.bitcast(w & hi_mask, jnp.float32))
                        for v, row in ((va, 2 * l), (vb, 2 * l + 1)):
                            m0 = v[0] + fx * (v[1] - v[0])
                            m1 = v[2] + fx * (v[3] - v[2])
                            m2 = v[4] + fx * (v[5] - v[4])
                            m3 = v[6] + fx * (v[7] - v[6])
                            n0 = m0 + fy * (m1 - m0)
                            n1 = m2 + fy * (m3 - m2)
                            acc_v[row, sl] = n0 + fz * (n1 - n0)
                    return c2

                lax.fori_loop(0, NGRP // 2, grp, 0)
                if local:
                    idx_cp(l2, gb2, j).start()
            pltpu.sync_copy(acc_v, out_hbm.at[:, pl.ds(p0, BLK)])
            return carry

        lax.fori_loop(0, NB, block_body, 0)

    return k(grid_packed, idx_all, pts_t)


def _mlp_body(pts_ref, emb_ref, w0_ref, w1_ref, c0_ref, c1_ref, c2_ref, out_ref):
    p = pts_ref[...]
    x = jnp.clip((p + 1.0) * 0.5, 0.0, 1.0 - 1e-6)
    centers = (lax.broadcasted_iota(jnp.int32, (NBINS, 1), 0).astype(jnp.float32)
               + 0.5)
    blobs = []
    for d in range(3):
        dd = x[d:d + 1, :] * jnp.float32(NBINS) - centers
        blobs.append(jnp.exp(-0.5 * dd * dd))
    hT = jnp.concatenate([emb_ref[...]] + blobs, axis=0)
    f32 = jnp.float32
    dn = (((0,), (0,)), ((), ()))
    h1 = jnp.maximum(
        lax.dot_general(w0_ref[...], hT, dn, preferred_element_type=f32), 0.0)
    h2 = lax.dot_general(w1_ref[...], h1, dn, preferred_element_type=f32)
    sdfT = h2[0:1, :]
    c = jnp.maximum(
        lax.dot_general(c0_ref[...], hT, dn, preferred_element_type=f32), 0.0)
    c = jnp.maximum(
        lax.dot_general(c1_ref[...], c, dn, preferred_element_type=f32), 0.0)
    rgbT = jax.nn.sigmoid(
        lax.dot_general(c2_ref[...], c, dn, preferred_element_type=f32))
    out_ref[...] = jnp.concatenate([rgbT, sdfT], axis=0)


def _tc_mlp(pts_t, embT, sdf_w0, sdf_w1, col_w0p, col_w1, col_w2):
    Bm = 4096
    full = lambda shape: pl.BlockSpec(shape, lambda i: tuple(0 for _ in shape))
    return pl.pallas_call(
        _mlp_body,
        grid=(N_PTS // Bm,),
        in_specs=[
            pl.BlockSpec((3, Bm), lambda i: (0, i)),
            pl.BlockSpec((2 * NLVL, Bm), lambda i: (0, i)),
            full(sdf_w0.shape),
            full(sdf_w1.shape),
            full(col_w0p.shape),
            full(col_w1.shape),
            full(col_w2.shape),
        ],
        out_specs=pl.BlockSpec((4, Bm), lambda i: (0, i)),
        out_shape=jax.ShapeDtypeStruct((4, N_PTS), jnp.float32),
    )(pts_t, embT, sdf_w0, sdf_w1, col_w0p, col_w1, col_w2)


def kernel(points, grid, sdf_w0, sdf_w1, col_w0, col_w1, col_w2):
    pts_t = points.T
    gu = lax.bitcast_convert_type(grid.astype(jnp.bfloat16),
                                  jnp.uint16).astype(jnp.uint32)
    grid_packed = lax.bitcast_convert_type(
        gu[..., 0] | (gu[..., 1] << 16), jnp.int32).reshape(NLVL * TBL)
    col_w0p = jnp.concatenate([col_w0[3 * NBINS:], col_w0[:3 * NBINS]], axis=0)

    idx_all = _tc_indices(pts_t)
    embT = _sc_embed(grid_packed, idx_all, pts_t)
    outT = _tc_mlp(pts_t, embT, sdf_w0, sdf_w1, col_w0p, col_w1, col_w2)
    return outT.T


# speedup vs baseline: 1.1032x; 1.1032x over previous
import functools

import jax
import jax.numpy as jnp
import numpy as np
from jax import lax
from jax.experimental import pallas as pl
from jax.experimental.pallas import tpu as pltpu
from jax.experimental.pallas import tpu_sc as plsc

N_PTS = 524288
NLVL = 16
LDIM = 2
TBL = 2 ** 19
NBINS = 16
_SCALE = float(np.exp2(np.log2(512 / 16) / (NLVL - 1)))
RES_LIST = [int(np.ceil(16 * _SCALE ** l)) for l in range(NLVL)]
TILED_LIST = [(r + 1) ** 3 <= TBL for r in RES_LIST]
HK1 = 2654435761
HK2 = 805459861
NW = 32
PW = N_PTS // NW
BLK = 1024
LOC_LVLS = 3
_loc_sz = [((RES_LIST[l] + 1) ** 3 + 15) & ~15 for l in range(LOC_LVLS)]
LOC_OFF = [sum(_loc_sz[:l]) for l in range(LOC_LVLS)]
LOC_TOTAL = sum(_loc_sz)


def _index_body(pts_ref, out_ref):
    p = pts_ref[...]
    x = jnp.clip((p + 1.0) * 0.5, 0.0, 1.0 - 1e-6)
    for l in range(NLVL):
        res = RES_LIST[l]
        pos = x * jnp.float32(res)
        p0 = jnp.floor(pos).astype(jnp.uint32)
        cx = p0[0:1, :]
        cy = p0[1:2, :]
        cz = p0[2:3, :]
        base = jnp.uint32(LOC_OFF[l] if l < LOC_LVLS else l * TBL)
        for c in range(8):
            if TILED_LIST[l]:
                s = res + 1
                ox, oy, oz = c & 1, (c >> 1) & 1, (c >> 2) & 1
                idx = ((cx + jnp.uint32(ox))
                       + (cy + jnp.uint32(oy)) * jnp.uint32(s)
                       + (cz + jnp.uint32(oz)) * jnp.uint32(s * s) + base)
            else:
                hx = cx + jnp.uint32(c & 1)
                hy = cy * jnp.uint32(HK1) + jnp.uint32(HK1 if (c >> 1) & 1 else 0)
                hz = cz * jnp.uint32(HK2) + jnp.uint32(HK2 if (c >> 2) & 1 else 0)
                idx = ((hx ^ hy ^ hz) & jnp.uint32(TBL - 1)) + base
            out_ref[pl.ds(l * 8 * BLK + c * BLK, BLK)] = (
                idx.astype(jnp.int32).reshape(BLK))


def _tc_indices(pts_t):
    return pl.pallas_call(
        _index_body,
        grid=(N_PTS // BLK,),
        in_specs=[pl.BlockSpec((3, BLK), lambda i: (0, i))],
        out_specs=pl.BlockSpec((NLVL * 8 * BLK,), lambda i: (i,)),
        out_shape=jax.ShapeDtypeStruct((N_PTS * NLVL * 8,), jnp.int32),
    )(pts_t)


def _sc_embed(grid_packed, idx_all, pts_t):
    mesh = plsc.VectorSubcoreMesh(core_axis_name="c", subcore_axis_name="s")
    NGRP = BLK // 16
    NB = PW // BLK

    @functools.partial(
        pl.kernel,
        mesh=mesh,
        compiler_params=pltpu.CompilerParams(
            needs_layout_passes=False, use_tc_tiling_on_sc=False),
        out_type=jax.ShapeDtypeStruct((2 * NLVL, N_PTS), jnp.float32),
        scratch_types=[
            [pltpu.VMEM((2 * BLK,), jnp.int32) for _ in range(8)],
            [pltpu.VMEM((2 * BLK,), jnp.int32) for _ in range(8)],
            pltpu.VMEM((3, BLK), jnp.float32),
            pltpu.VMEM((2 * NLVL, BLK), jnp.float32),
            pltpu.VMEM((LOC_TOTAL,), jnp.int32),
            [pltpu.SemaphoreType.DMA for _ in range(2)],
            [pltpu.SemaphoreType.DMA for _ in range(2)],
        ],
    )
    def k(grid_hbm, idx_hbm, pts_hbm, out_hbm, idxb, crnb, xv, acc_v,
          tab_v, isem, gsem):
        wid = lax.axis_index("s") * 2 + lax.axis_index("c")
        base_pt = wid * PW
        wb0 = wid * NB
        hi_mask = jnp.int32(-65536)

        class _Multi:
            def __init__(self, hs):
                self.hs = hs

            def start(self):
                for h in self.hs:
                    h.start()

            def wait(self):
                for h in self.hs:
                    h.wait()

        def idx_cp(l, gb, j):
            return _Multi([
                pltpu.make_async_copy(
                    idx_hbm.at[pl.ds(gb * (NLVL * 8 * BLK) + l * 8 * BLK
                                     + q * 2 * BLK, 2 * BLK)],
                    idxb[j * 4 + q], isem[j])
                for q in range(4)
            ])

        def gath(j):
            return _Multi([
                pltpu.make_async_copy(grid_hbm.at[idxb[j * 4 + q]],
                                      crnb[j * 4 + q], gsem[j])
                for q in range(4)
            ])

        idx_cp(0, wb0, 0).start()
        idx_cp(1, wb0, 1).start()
        for l in range(LOC_LVLS):
            pltpu.sync_copy(grid_hbm.at[pl.ds(l * TBL, _loc_sz[l])],
                            tab_v.at[pl.ds(LOC_OFF[l], _loc_sz[l])])
        idx_cp(0, wb0, 0).wait()

        def block_body(b, carry):
            p0 = base_pt + b * BLK
            gb = wb0 + b
            not_last = b < NB - 1
            pltpu.sync_copy(pts_hbm.at[:, pl.ds(p0, BLK)], xv)
            for l in range(NLVL):
                j = l & 1
                jn = 1 - j
                local = l < LOC_LVLS
                res_f = jnp.float32(RES_LIST[l])
                if not local:
                    gath(j).wait()
                l1 = (l + 1) % NLVL
                gb1 = gb + (l + 1) // NLVL
                if l < NLVL - 1:
                    idx_cp(l1, gb1, jn).wait()
                    if l1 >= LOC_LVLS:
                        gath(jn).start()
                else:
                    @pl.when(not_last)
                    def _():
                        idx_cp(l1, gb1, jn).wait()
                l2 = (l + 2) % NLVL
                gb2 = gb + (l + 2) // NLVL
                if not local:
                    if l < NLVL - 2:
                        idx_cp(l2, gb2, j).start()
                    else:
                        @pl.when(not_last)
                        def _():
                            idx_cp(l2, gb2, j).start()

                crn4 = [crnb[j * 4 + q] for q in range(4)]
                idx4 = [idxb[j * 4 + q] for q in range(4)]

                def grp(g, c2):
                    for u in range(2):
                        sl = pl.ds((2 * g + u) * 16, 16)
                        fr = []
                        for d in range(3):
                            xe = xv[d, sl]
                            pos = jnp.minimum(
                                jnp.maximum(xe * 0.5 + 0.5, 0.0),
                                1.0 - 1e-6) * res_f
                            fr.append(pos - lax.convert_element_type(
                                lax.convert_element_type(pos, jnp.int32),
                                jnp.float32))
                        fx, fy, fz = fr
                        va = []
                        vb = []
                        for c in range(8):
                            if local:
                                iv = idx4[c // 2][
                                    pl.ds((c & 1) * BLK + (2 * g + u) * 16, 16)]
                                w = plsc.load_gather(tab_v, [iv])
                            else:
                                w = crn4[c // 2][
                                    pl.ds((c & 1) * BLK + (2 * g + u) * 16, 16)]
                            va.append(plsc.bitcast(lax.shift_left(w, 16),
                                                   jnp.float32))
                            vb.append(plsc.bitcast(w & hi_mask, jnp.float32))
                        for v, row in ((va, 2 * l), (vb, 2 * l + 1)):
                            m0 = v[0] + fx * (v[1] - v[0])
                            m1 = v[2] + fx * (v[3] - v[2])
                            m2 = v[4] + fx * (v[5] - v[4])
                            m3 = v[6] + fx * (v[7] - v[6])
                            n0 = m0 + fy * (m1 - m0)
                            n1 = m2 + fy * (m3 - m2)
                            acc_v[row, sl] = n0 + fz * (n1 - n0)
                    return c2

                lax.fori_loop(0, NGRP // 2, grp, 0)
                if local:
                    idx_cp(l2, gb2, j).start()
            pltpu.sync_copy(acc_v, out_hbm.at[:, pl.ds(p0, BLK)])
            return carry

        lax.fori_loop(0, NB, block_body, 0)

    return k(grid_packed, idx_all, pts_t)


def _mlp_body(pts_ref, emb_ref, w0_ref, w1_ref, c0_ref, c1_ref, c2_ref, out_ref):
    p = pts_ref[...]
    x = jnp.clip((p + 1.0) * 0.5, 0.0, 1.0 - 1e-6)
    centers = (lax.broadcasted_iota(jnp.int32, (NBINS, 1), 0).astype(jnp.float32)
               + 0.5)
    blobs = []
    for d in range(3):
        dd = x[d:d + 1, :] * jnp.float32(NBINS) - centers
        blobs.append(jnp.exp(-0.5 * dd * dd))
    hT = jnp.concatenate([emb_ref[...]] + blobs, axis=0)
    f32 = jnp.float32
    dn = (((0,), (0,)), ((), ()))
    h1 = jnp.maximum(
        lax.dot_general(w0_ref[...], hT, dn, preferred_element_type=f32), 0.0)
    h2 = lax.dot_general(w1_ref[...], h1, dn, preferred_element_type=f32)
    sdfT = h2[0:1, :]
    c = jnp.maximum(
        lax.dot_general(c0_ref[...], hT, dn, preferred_element_type=f32), 0.0)
    c = jnp.maximum(
        lax.dot_general(c1_ref[...], c, dn, preferred_element_type=f32), 0.0)
    rgbT = jax.nn.sigmoid(
        lax.dot_general(c2_ref[...], c, dn, preferred_element_type=f32))
    out_ref[...] = jnp.concatenate([rgbT, sdfT], axis=0)


def _tc_mlp(pts_t, embT, sdf_w0, sdf_w1, col_w0p, col_w1, col_w2):
    Bm = 4096
    full = lambda shape: pl.BlockSpec(shape, lambda i: tuple(0 for _ in shape))
    return pl.pallas_call(
        _mlp_body,
        grid=(N_PTS // Bm,),
        in_specs=[
            pl.BlockSpec((3, Bm), lambda i: (0, i)),
            pl.BlockSpec((2 * NLVL, Bm), lambda i: (0, i)),
            full(sdf_w0.shape),
            full(sdf_w1.shape),
            full(col_w0p.shape),
            full(col_w1.shape),
            full(col_w2.shape),
        ],
        out_specs=pl.BlockSpec((4, Bm), lambda i: (0, i)),
        out_shape=jax.ShapeDtypeStruct((4, N_PTS), jnp.float32),
    )(pts_t, embT, sdf_w0, sdf_w1, col_w0p, col_w1, col_w2)


def kernel(points, grid, sdf_w0, sdf_w1, col_w0, col_w1, col_w2):
    pts_t = points.T
    gu = lax.bitcast_convert_type(grid.astype(jnp.bfloat16),
                                  jnp.uint16).astype(jnp.uint32)
    grid_packed = lax.bitcast_convert_type(
        gu[..., 0] | (gu[..., 1] << 16), jnp.int32).reshape(NLVL * TBL)
    col_w0p = jnp.concatenate([col_w0[3 * NBINS:], col_w0[:3 * NBINS]], axis=0)

    idx_all = _tc_indices(pts_t)
    embT = _sc_embed(grid_packed, idx_all, pts_t)
    outT = _tc_mlp(pts_t, embT, sdf_w0, sdf_w1, col_w0p, col_w1, col_w2)
    return outT.T
